# baseline (speedup 1.0000x reference)
import functools

import jax
import jax.numpy as jnp
from jax import lax
from jax.experimental import pallas as pl
from jax.experimental.pallas import tpu as pltpu
from jax.experimental.pallas import tpu_sc as plsc

NC = 2
NS = 16
NW = NC * NS
CHUNK = 128
NBI = 8
LANES = 16


def _spmm_partials(v, ids, wts, zeros, n_super, n, d):
    rows_per_sub = n // NS
    n_chunks = n_super * NBI
    mesh = plsc.VectorSubcoreMesh(core_axis_name="c", subcore_axis_name="s")

    @functools.partial(
        pl.kernel,
        out_type=jax.ShapeDtypeStruct((NC, n, d), jnp.float32),
        mesh=mesh,
        scratch_types=[
            pltpu.VMEM((NBI, 2, CHUNK), jnp.int32),
            pltpu.VMEM((NBI, 2, CHUNK), jnp.int32),
            pltpu.VMEM((NBI, CHUNK), jnp.float32),
            pltpu.VMEM((NBI, CHUNK), jnp.float32),
            pltpu.VMEM((2, CHUNK, d), jnp.float32),
            pltpu.VMEM_SHARED((n, d), jnp.float32),
            [pltpu.SemaphoreType.DMA] * 2,
            [pltpu.SemaphoreType.DMA] * 2,
            [pltpu.SemaphoreType.DMA] * 2,
        ],
    )
    def k(v_hbm, ids_hbm, w_hbm, z_hbm, out_hbm,
          slot_a, slot_b, wslot_a, wslot_b, rows, acc, isems, gsems, ssems):
        cid = lax.axis_index("c")
        sid = lax.axis_index("s")
        wid = cid * NS + sid
        row0 = sid * rows_per_sub
        slots = (slot_a, slot_b)
        wslots = (wslot_a, wslot_b)

        def idx_start(s, sl):
            pltpu.async_copy(ids_hbm.at[wid, s], slots[sl], isems[sl])
            pltpu.async_copy(w_hbm.at[wid, s], wslots[sl], isems[sl])

        def idx_wait(s, sl):
            pltpu.make_async_copy(ids_hbm.at[wid, s], slots[sl],
                                  isems[sl]).wait()
            pltpu.make_async_copy(w_hbm.at[wid, s], wslots[sl],
                                  isems[sl]).wait()

        def gather_start(sl, j, b):
            pltpu.async_copy(v_hbm.at[slots[sl].at[j, 0]], rows.at[b],
                             gsems[b])

        def gather_wait(sl, j, b):
            pltpu.make_async_copy(v_hbm.at[slots[sl].at[j, 0]], rows.at[b],
                                  gsems[b]).wait()

        def scatter_start(sl, j, b):
            pltpu.async_copy(rows.at[b], acc.at[slots[sl].at[j, 1]],
                             ssems[b], add=True)

        def scatter_wait(sl, j, b):
            pltpu.make_async_copy(rows.at[b], acc.at[slots[sl].at[j, 1]],
                                  ssems[b]).wait()

        def compute(sl, j, b):
            def group(g, carry):
                wvec = wslots[sl][j, pl.ds(g * LANES, LANES)]
                for i in range(LANES):
                    wgt = wvec[i]
                    e = g * LANES + i
                    for jj in range(d // LANES):
                        cols = pl.ds(jj * LANES, LANES)
                        rows[b, e, cols] = rows[b, e, cols] * wgt
                return carry

            lax.fori_loop(0, CHUNK // LANES, group, 0)

        idx_start(0, 0)
        idx_wait(0, 0)
        gather_start(0, 0, 0)
        pltpu.sync_copy(z_hbm.at[pl.ds(row0, rows_per_sub)],
                        acc.at[pl.ds(row0, rows_per_sub)])
        plsc.subcore_barrier()

        def body(i, carry):
            for sp in range(2):
                s = 2 * i + sp
                for j in range(NBI):
                    c = s * NBI + j
                    b = j % 2
                    gather_wait(sp, j, b)

                    psl, pj = (sp, j - 1) if j else (1 - sp, NBI - 1)

                    @pl.when(c >= 1)
                    def _():
                        scatter_wait(psl, pj, 1 - b)

                    if j == 0:
                        @pl.when(s + 1 < n_super)
                        def _():
                            idx_start(s + 1, 1 - sp)

                    if j == NBI - 1:
                        @pl.when(s + 1 < n_super)
                        def _():
                            idx_wait(s + 1, 1 - sp)

                    nsl, nj = (sp, j + 1) if j < NBI - 1 else (1 - sp, 0)

                    @pl.when(c + 1 < n_chunks)
                    def _():
                        gather_start(nsl, nj, 1 - b)

                    compute(sp, j, b)
                    scatter_start(sp, j, b)
            return carry

        lax.fori_loop(0, n_super // 2, body, 0)
        scatter_wait(1, NBI - 1, 1)
        plsc.subcore_barrier()
        pltpu.sync_copy(acc.at[pl.ds(row0, rows_per_sub)],
                        out_hbm.at[cid, pl.ds(row0, rows_per_sub)])

    return k(v, ids, wts, zeros)


def _combine_tc(x, p):
    def body(x_ref, p_ref, y_ref):
        y_ref[...] = x_ref[...] + 2.0 * (p_ref[0] + p_ref[1])

    return pl.pallas_call(
        body, out_shape=jax.ShapeDtypeStruct(x.shape, jnp.float32))(x, p)


def _matmul_tc(q, w_mat, bias2d):
    def body(q_ref, w_ref, b_ref, o_ref):
        s = q_ref[0] + q_ref[1]
        o_ref[...] = jnp.dot(s, w_ref[...],
                             preferred_element_type=jnp.float32) + b_ref[...]

    n, d = q.shape[1], q.shape[2]
    return pl.pallas_call(
        body, out_shape=jax.ShapeDtypeStruct((n, d), jnp.float32))(q, w_mat, bias2d)


def kernel(x, edge_index, edge_weight, W, bias):
    n, d = x.shape
    src = edge_index[0]
    dst = edge_index[1]
    e = src.shape[0]
    per_super2 = NW * CHUNK * NBI * 2
    n_super = 2 * (-(-e // per_super2))
    e_pad = NW * CHUNK * NBI * n_super // 2 * 2
    e_pad = NW * CHUNK * NBI * n_super
    pad = e_pad - e
    if pad:
        src = jnp.pad(src, (0, pad))
        dst = jnp.pad(dst, (0, pad))
        edge_weight = jnp.pad(edge_weight, (0, pad))
    shape4 = (NW, n_super, NBI, CHUNK)
    ids = jnp.stack([src.reshape(shape4), dst.reshape(shape4)], axis=3)
    wts = edge_weight.reshape(shape4)
    n_pad = -(-n // (NS * 8)) * (NS * 8)
    x_pad = jnp.pad(x, ((0, n_pad - n), (0, 0))) if n_pad != n else x
    zeros = jnp.zeros_like(x_pad)
    p = _spmm_partials(x_pad, ids, wts, zeros, n_super, n_pad, d)
    y = _combine_tc(x_pad, p)
    q = _spmm_partials(y, ids, wts, zeros, n_super, n_pad, d)
    return _matmul_tc(q, W, bias.reshape(1, d))[:n]
